# Initial kernel scaffold; baseline (speedup 1.0000x reference)
import jax
import jax.numpy as jnp
from jax.experimental import pallas as pl


def kernel(x, w1, g1, b1, w2, g2, b2, w3, g3, b3, w_sc, g_sc, b_sc, w_fc1, w_fc2):
    raise NotImplementedError("write your pallas kernel here")


# speedup vs baseline: 2.2099x; 2.2099x over previous
import functools

import jax
import jax.numpy as jnp
from jax.experimental import pallas as pl
from jax.experimental.pallas import tpu as pltpu

_EPS = 1e-5


def _mm_stats_kernel(a_ref, b_ref, out_ref, stats_ref):
    acc = jnp.dot(a_ref[...], b_ref[...], preferred_element_type=jnp.float32)
    out_ref[...] = acc.astype(out_ref.dtype)
    s0 = jnp.sum(acc, axis=0, keepdims=True)
    s1 = jnp.sum(acc * acc, axis=0, keepdims=True)
    stats_ref[...] = jnp.concatenate([s0, s1], axis=0)[None]


def _conv3_sc_kernel(a1_ref, s2_ref, h2_ref, a2_ref, w3_ref, wsc_ref,
                     y3_ref, ysc_ref, st3_ref, stsc_ref):
    a1 = a1_ref[...].astype(jnp.float32) * s2_ref[...] + h2_ref[...]
    a1 = jnp.maximum(a1, 0.0).astype(jnp.bfloat16)
    acc3 = jnp.dot(a1, w3_ref[...], preferred_element_type=jnp.float32)
    y3_ref[...] = acc3.astype(y3_ref.dtype)
    st3_ref[...] = jnp.concatenate(
        [jnp.sum(acc3, axis=0, keepdims=True),
         jnp.sum(acc3 * acc3, axis=0, keepdims=True)], axis=0)[None]

    accs = jnp.dot(a2_ref[...], wsc_ref[...], preferred_element_type=jnp.float32)
    ysc_ref[...] = accs.astype(ysc_ref.dtype)
    stsc_ref[...] = jnp.concatenate(
        [jnp.sum(accs, axis=0, keepdims=True),
         jnp.sum(accs * accs, axis=0, keepdims=True)], axis=0)[None]


def _final_kernel(y3_ref, ysc_ref, pool_ref, wf1_ref, wf2_ref,
                  s3_ref, h3_ref, ssc_ref, hsc_ref, out_ref):
    g = jnp.dot(pool_ref[0], wf1_ref[...], preferred_element_type=jnp.float32)
    g = jnp.maximum(g, 0.0)
    g = jnp.dot(g, wf2_ref[...], preferred_element_type=jnp.float32)
    g = 1.0 / (1.0 + jnp.exp(-g))

    y = y3_ref[0].astype(jnp.float32) * s3_ref[...] + h3_ref[...]
    sc = ysc_ref[0].astype(jnp.float32) * ssc_ref[...] + hsc_ref[...]
    out_ref[0] = jnp.maximum(y * g + sc, 0.0)


def _matmul_stats(a, b, tm):
    m, k = a.shape
    n = b.shape[1]
    nm = m // tm
    return pl.pallas_call(
        _mm_stats_kernel,
        grid=(nm,),
        in_specs=[
            pl.BlockSpec((tm, k), lambda i: (i, 0)),
            pl.BlockSpec((k, n), lambda i: (0, 0)),
        ],
        out_specs=[
            pl.BlockSpec((tm, n), lambda i: (i, 0)),
            pl.BlockSpec((1, 2, n), lambda i: (i, 0, 0)),
        ],
        out_shape=(
            jax.ShapeDtypeStruct((m, n), jnp.bfloat16),
            jax.ShapeDtypeStruct((nm, 2, n), jnp.float32),
        ),
        compiler_params=pltpu.CompilerParams(
            dimension_semantics=("parallel",)),
    )(a, b)


def _bn_fold(stats, m_true, gamma, beta):
    tot = jnp.sum(stats, axis=0)
    mean = tot[0] / float(m_true)
    var = jnp.maximum(tot[1] / float(m_true) - mean * mean, 0.0)
    inv_std = jax.lax.rsqrt(var + _EPS)
    scale = gamma.astype(jnp.float32) * inv_std
    shift = beta.astype(jnp.float32) - mean * scale
    return scale, shift


def kernel(x, w1, g1, b1, w2, g2, b2, w3, g3, b3, w_sc, g_sc, b_sc,
           w_fc1, w_fc2):
    n, cin, h, w = x.shape
    c_mid = w1.shape[0]
    c_out = w3.shape[0]
    hid = w_fc1.shape[0]
    oh, ow = h // 2, w // 2
    m1 = n * h * w
    m2 = n * oh * ow
    mb = oh * ow

    x_nhwc = jnp.transpose(x, (0, 2, 3, 1)).astype(jnp.bfloat16)
    a1 = x_nhwc.reshape(m1, cin)
    w1m = jnp.transpose(w1.reshape(c_mid, cin)).astype(jnp.bfloat16)
    raw1, st1 = _matmul_stats(a1, w1m, tm=512 if m1 % 512 == 0 else m1)
    sc1, sh1 = _bn_fold(st1, m1, g1, b1)

    xn = raw1.reshape(n, h, w, c_mid).astype(jnp.float32) * sc1 + sh1
    xn = jnp.maximum(xn, 0.0).astype(jnp.bfloat16)
    xp = jnp.pad(xn, ((0, 0), (1, 1), (1, 1), (0, 0)))
    cols = []
    for i in range(3):
        for j in range(3):
            cols.append(jax.lax.slice(
                xp, (0, i, j, 0), (n, i + h - 1, j + w - 1, c_mid),
                (1, 2, 2, 1)))
    patches = jnp.stack(cols, axis=3).reshape(m2, 9 * c_mid)
    patches = jnp.pad(patches, ((0, 0), (0, 64)))
    w2m = jnp.transpose(w2, (2, 3, 1, 0)).reshape(9 * c_mid, c_mid)
    w2m = jnp.pad(w2m, ((0, 64), (0, 0))).astype(jnp.bfloat16)
    raw2, st2 = _matmul_stats(patches, w2m, tm=mb)
    sc2, sh2 = _bn_fold(st2, m2, g2, b2)

    x_sc = x_nhwc[:, ::2, ::2, :].reshape(m2, cin)
    w3m = jnp.transpose(w3.reshape(c_out, c_mid)).astype(jnp.bfloat16)
    wscm = jnp.transpose(w_sc.reshape(c_out, cin)).astype(jnp.bfloat16)
    raw3, rawsc, st3, stsc = pl.pallas_call(
        _conv3_sc_kernel,
        grid=(n,),
        in_specs=[
            pl.BlockSpec((mb, c_mid), lambda i: (i, 0)),
            pl.BlockSpec((1, c_mid), lambda i: (0, 0)),
            pl.BlockSpec((1, c_mid), lambda i: (0, 0)),
            pl.BlockSpec((mb, cin), lambda i: (i, 0)),
            pl.BlockSpec((c_mid, c_out), lambda i: (0, 0)),
            pl.BlockSpec((cin, c_out), lambda i: (0, 0)),
        ],
        out_specs=[
            pl.BlockSpec((mb, c_out), lambda i: (i, 0)),
            pl.BlockSpec((mb, c_out), lambda i: (i, 0)),
            pl.BlockSpec((1, 2, c_out), lambda i: (i, 0, 0)),
            pl.BlockSpec((1, 2, c_out), lambda i: (i, 0, 0)),
        ],
        out_shape=(
            jax.ShapeDtypeStruct((m2, c_out), jnp.bfloat16),
            jax.ShapeDtypeStruct((m2, c_out), jnp.bfloat16),
            jax.ShapeDtypeStruct((n, 2, c_out), jnp.float32),
            jax.ShapeDtypeStruct((n, 2, c_out), jnp.float32),
        ),
        compiler_params=pltpu.CompilerParams(
            dimension_semantics=("parallel",)),
    )(raw2, sc2.reshape(1, c_mid), sh2.reshape(1, c_mid), x_sc, w3m, wscm)

    sc3, sh3 = _bn_fold(st3, m2, g3, b3)
    sccut, shcut = _bn_fold(stsc, m2, g_sc, b_sc)

    pooled = st3[:, 0, :] / float(mb) * sc3 + sh3
    wf1 = jnp.pad(jnp.transpose(w_fc1), ((0, 0), (0, 128 - hid)))
    wf2 = jnp.pad(jnp.transpose(w_fc2), ((0, 128 - hid), (0, 0)))

    out = pl.pallas_call(
        _final_kernel,
        grid=(n,),
        in_specs=[
            pl.BlockSpec((1, mb, c_out), lambda i: (i, 0, 0)),
            pl.BlockSpec((1, mb, c_out), lambda i: (i, 0, 0)),
            pl.BlockSpec((1, 1, c_out), lambda i: (i, 0, 0)),
            pl.BlockSpec((cin, 128), lambda i: (0, 0)),
            pl.BlockSpec((128, c_out), lambda i: (0, 0)),
            pl.BlockSpec((1, c_out), lambda i: (0, 0)),
            pl.BlockSpec((1, c_out), lambda i: (0, 0)),
            pl.BlockSpec((1, c_out), lambda i: (0, 0)),
            pl.BlockSpec((1, c_out), lambda i: (0, 0)),
        ],
        out_specs=pl.BlockSpec((1, mb, c_out), lambda i: (i, 0, 0)),
        out_shape=jax.ShapeDtypeStruct((n, mb, c_out), jnp.float32),
        compiler_params=pltpu.CompilerParams(
            dimension_semantics=("parallel",)),
    )(raw3.reshape(n, mb, c_out), rawsc.reshape(n, mb, c_out),
      pooled.reshape(n, 1, c_out), wf1, wf2,
      sc3.reshape(1, c_out), sh3.reshape(1, c_out),
      sccut.reshape(1, c_out), shcut.reshape(1, c_out))

    return jnp.transpose(out, (0, 2, 1)).reshape(n, c_out, oh, ow)


# speedup vs baseline: 2.5684x; 1.1622x over previous
import functools

import jax
import jax.numpy as jnp
from jax.experimental import pallas as pl
from jax.experimental.pallas import tpu as pltpu

_EPS = 1e-5


def _mm_stats_kernel(a_ref, b_ref, out_ref, stats_ref):
    acc = jnp.dot(a_ref[...], b_ref[...], preferred_element_type=jnp.float32)
    out_ref[...] = acc.astype(out_ref.dtype)
    s0 = jnp.sum(acc, axis=0, keepdims=True)
    s1 = jnp.sum(acc * acc, axis=0, keepdims=True)
    stats_ref[...] = jnp.concatenate([s0, s1], axis=0)[None]


def _conv2_kernel(xps_ref, w2_ref, out_ref, stats_ref, *, oh, mb):
    blk = xps_ref[0]
    acc = jnp.zeros((mb, blk.shape[-1]), jnp.float32)
    for i in range(3):
        for j in range(3):
            a = blk[i % 2, j % 2, i // 2:i // 2 + oh, j // 2:j // 2 + oh, :]
            acc += jnp.dot(a.reshape(mb, -1), w2_ref[3 * i + j],
                           preferred_element_type=jnp.float32)
    out_ref[...] = acc.astype(out_ref.dtype)
    stats_ref[...] = jnp.concatenate(
        [jnp.sum(acc, axis=0, keepdims=True),
         jnp.sum(acc * acc, axis=0, keepdims=True)], axis=0)[None]


def _conv3_sc_kernel(a1_ref, s2_ref, h2_ref, a2_ref, w3_ref, wsc_ref,
                     y3_ref, ysc_ref, st3_ref, stsc_ref):
    a1 = a1_ref[...].astype(jnp.float32) * s2_ref[...] + h2_ref[...]
    a1 = jnp.maximum(a1, 0.0).astype(jnp.bfloat16)
    acc3 = jnp.dot(a1, w3_ref[...], preferred_element_type=jnp.float32)
    y3_ref[...] = acc3.astype(y3_ref.dtype)
    st3_ref[...] = jnp.concatenate(
        [jnp.sum(acc3, axis=0, keepdims=True),
         jnp.sum(acc3 * acc3, axis=0, keepdims=True)], axis=0)[None]

    accs = jnp.dot(a2_ref[...], wsc_ref[...], preferred_element_type=jnp.float32)
    ysc_ref[...] = accs.astype(ysc_ref.dtype)
    stsc_ref[...] = jnp.concatenate(
        [jnp.sum(accs, axis=0, keepdims=True),
         jnp.sum(accs * accs, axis=0, keepdims=True)], axis=0)[None]


def _final_kernel(y3_ref, ysc_ref, pool_ref, wf1_ref, wf2_ref,
                  s3_ref, h3_ref, ssc_ref, hsc_ref, out_ref):
    g = jnp.dot(pool_ref[0], wf1_ref[...], preferred_element_type=jnp.float32)
    g = jnp.maximum(g, 0.0)
    g = jnp.dot(g, wf2_ref[...], preferred_element_type=jnp.float32)
    g = 1.0 / (1.0 + jnp.exp(-g))

    y = y3_ref[0].astype(jnp.float32) * s3_ref[...] + h3_ref[...]
    sc = ysc_ref[0].astype(jnp.float32) * ssc_ref[...] + hsc_ref[...]
    out_ref[0] = jnp.transpose(jnp.maximum(y * g + sc, 0.0))


def _matmul_stats(a, b, tm):
    m, k = a.shape
    n = b.shape[1]
    nm = m // tm
    return pl.pallas_call(
        _mm_stats_kernel,
        grid=(nm,),
        in_specs=[
            pl.BlockSpec((tm, k), lambda i: (i, 0)),
            pl.BlockSpec((k, n), lambda i: (0, 0)),
        ],
        out_specs=[
            pl.BlockSpec((tm, n), lambda i: (i, 0)),
            pl.BlockSpec((1, 2, n), lambda i: (i, 0, 0)),
        ],
        out_shape=(
            jax.ShapeDtypeStruct((m, n), jnp.bfloat16),
            jax.ShapeDtypeStruct((nm, 2, n), jnp.float32),
        ),
        compiler_params=pltpu.CompilerParams(
            dimension_semantics=("parallel",)),
    )(a, b)


def _bn_fold(stats, m_true, gamma, beta):
    tot = jnp.sum(stats, axis=0)
    mean = tot[0] / float(m_true)
    var = jnp.maximum(tot[1] / float(m_true) - mean * mean, 0.0)
    inv_std = jax.lax.rsqrt(var + _EPS)
    scale = gamma.astype(jnp.float32) * inv_std
    shift = beta.astype(jnp.float32) - mean * scale
    return scale, shift


def kernel(x, w1, g1, b1, w2, g2, b2, w3, g3, b3, w_sc, g_sc, b_sc,
           w_fc1, w_fc2):
    n, cin, h, w = x.shape
    c_mid = w1.shape[0]
    c_out = w3.shape[0]
    hid = w_fc1.shape[0]
    oh, ow = h // 2, w // 2
    m1 = n * h * w
    m2 = n * oh * ow
    mb = oh * ow

    x_nhwc = jnp.transpose(x, (0, 2, 3, 1)).astype(jnp.bfloat16)
    a1 = x_nhwc.reshape(m1, cin)
    w1m = jnp.transpose(w1.reshape(c_mid, cin)).astype(jnp.bfloat16)
    raw1, st1 = _matmul_stats(a1, w1m, tm=512 if m1 % 512 == 0 else m1)
    sc1, sh1 = _bn_fold(st1, m1, g1, b1)

    hp = h // 2 + 1
    xn = raw1.reshape(n, h, w, c_mid).astype(jnp.float32) * sc1 + sh1
    xn = jnp.maximum(xn, 0.0).astype(jnp.bfloat16)
    xp = jnp.pad(xn, ((0, 0), (1, 1), (1, 1), (0, 0)))
    xps = xp.reshape(n, hp, 2, hp, 2, c_mid).transpose(0, 2, 4, 1, 3, 5)
    w2m = jnp.transpose(w2, (2, 3, 1, 0)).reshape(9, c_mid, c_mid)
    w2m = w2m.astype(jnp.bfloat16)
    raw2, st2 = pl.pallas_call(
        functools.partial(_conv2_kernel, oh=oh, mb=mb),
        grid=(n,),
        in_specs=[
            pl.BlockSpec((1, 2, 2, hp, hp, c_mid),
                         lambda i: (i, 0, 0, 0, 0, 0)),
            pl.BlockSpec((9, c_mid, c_mid), lambda i: (0, 0, 0)),
        ],
        out_specs=[
            pl.BlockSpec((mb, c_mid), lambda i: (i, 0)),
            pl.BlockSpec((1, 2, c_mid), lambda i: (i, 0, 0)),
        ],
        out_shape=(
            jax.ShapeDtypeStruct((m2, c_mid), jnp.bfloat16),
            jax.ShapeDtypeStruct((n, 2, c_mid), jnp.float32),
        ),
        compiler_params=pltpu.CompilerParams(
            dimension_semantics=("parallel",)),
    )(xps, w2m)
    sc2, sh2 = _bn_fold(st2, m2, g2, b2)

    x_sc = x_nhwc[:, ::2, ::2, :].reshape(m2, cin)
    w3m = jnp.transpose(w3.reshape(c_out, c_mid)).astype(jnp.bfloat16)
    wscm = jnp.transpose(w_sc.reshape(c_out, cin)).astype(jnp.bfloat16)
    raw3, rawsc, st3, stsc = pl.pallas_call(
        _conv3_sc_kernel,
        grid=(n,),
        in_specs=[
            pl.BlockSpec((mb, c_mid), lambda i: (i, 0)),
            pl.BlockSpec((1, c_mid), lambda i: (0, 0)),
            pl.BlockSpec((1, c_mid), lambda i: (0, 0)),
            pl.BlockSpec((mb, cin), lambda i: (i, 0)),
            pl.BlockSpec((c_mid, c_out), lambda i: (0, 0)),
            pl.BlockSpec((cin, c_out), lambda i: (0, 0)),
        ],
        out_specs=[
            pl.BlockSpec((mb, c_out), lambda i: (i, 0)),
            pl.BlockSpec((mb, c_out), lambda i: (i, 0)),
            pl.BlockSpec((1, 2, c_out), lambda i: (i, 0, 0)),
            pl.BlockSpec((1, 2, c_out), lambda i: (i, 0, 0)),
        ],
        out_shape=(
            jax.ShapeDtypeStruct((m2, c_out), jnp.bfloat16),
            jax.ShapeDtypeStruct((m2, c_out), jnp.bfloat16),
            jax.ShapeDtypeStruct((n, 2, c_out), jnp.float32),
            jax.ShapeDtypeStruct((n, 2, c_out), jnp.float32),
        ),
        compiler_params=pltpu.CompilerParams(
            dimension_semantics=("parallel",)),
    )(raw2, sc2.reshape(1, c_mid), sh2.reshape(1, c_mid), x_sc, w3m, wscm)

    sc3, sh3 = _bn_fold(st3, m2, g3, b3)
    sccut, shcut = _bn_fold(stsc, m2, g_sc, b_sc)

    pooled = st3[:, 0, :] / float(mb) * sc3 + sh3
    wf1 = jnp.pad(jnp.transpose(w_fc1), ((0, 0), (0, 128 - hid)))
    wf2 = jnp.pad(jnp.transpose(w_fc2), ((0, 128 - hid), (0, 0)))

    out = pl.pallas_call(
        _final_kernel,
        grid=(n,),
        in_specs=[
            pl.BlockSpec((1, mb, c_out), lambda i: (i, 0, 0)),
            pl.BlockSpec((1, mb, c_out), lambda i: (i, 0, 0)),
            pl.BlockSpec((1, 1, c_out), lambda i: (i, 0, 0)),
            pl.BlockSpec((cin, 128), lambda i: (0, 0)),
            pl.BlockSpec((128, c_out), lambda i: (0, 0)),
            pl.BlockSpec((1, c_out), lambda i: (0, 0)),
            pl.BlockSpec((1, c_out), lambda i: (0, 0)),
            pl.BlockSpec((1, c_out), lambda i: (0, 0)),
            pl.BlockSpec((1, c_out), lambda i: (0, 0)),
        ],
        out_specs=pl.BlockSpec((1, c_out, mb), lambda i: (i, 0, 0)),
        out_shape=jax.ShapeDtypeStruct((n, c_out, mb), jnp.float32),
        compiler_params=pltpu.CompilerParams(
            dimension_semantics=("parallel",)),
    )(raw3.reshape(n, mb, c_out), rawsc.reshape(n, mb, c_out),
      pooled.reshape(n, 1, c_out), wf1, wf2,
      sc3.reshape(1, c_out), sh3.reshape(1, c_out),
      sccut.reshape(1, c_out), shcut.reshape(1, c_out))

    return out.reshape(n, c_out, oh, ow)


# speedup vs baseline: 2.7519x; 1.0715x over previous
import functools

import jax
import jax.numpy as jnp
from jax.experimental import pallas as pl
from jax.experimental.pallas import tpu as pltpu

_EPS = 1e-5


def _conv1_kernel(x_ref, b_ref, out_ref, stats_ref):
    acc = jax.lax.dot_general(
        x_ref[0].astype(jnp.bfloat16), b_ref[...],
        dimension_numbers=(((0,), (0,)), ((), ())),
        preferred_element_type=jnp.float32)
    out_ref[...] = acc.astype(out_ref.dtype)
    stats_ref[...] = jnp.concatenate(
        [jnp.sum(acc, axis=0, keepdims=True),
         jnp.sum(acc * acc, axis=0, keepdims=True)], axis=0)[None]


def _conv2_kernel(xps_ref, w2_ref, out_ref, stats_ref, *, oh, mb):
    blk = xps_ref[0]
    acc = jnp.zeros((mb, blk.shape[-1]), jnp.float32)
    for i in range(3):
        for j in range(3):
            a = blk[i % 2, j % 2, i // 2:i // 2 + oh, j // 2:j // 2 + oh, :]
            acc += jnp.dot(a.reshape(mb, -1), w2_ref[3 * i + j],
                           preferred_element_type=jnp.float32)
    out_ref[...] = acc.astype(out_ref.dtype)
    stats_ref[...] = jnp.concatenate(
        [jnp.sum(acc, axis=0, keepdims=True),
         jnp.sum(acc * acc, axis=0, keepdims=True)], axis=0)[None]


def _conv3_sc_kernel(a1_ref, s2_ref, h2_ref, a2_ref, w3_ref, wsc_ref,
                     y3_ref, ysc_ref, st3_ref, stsc_ref):
    a1 = a1_ref[...].astype(jnp.float32) * s2_ref[...] + h2_ref[...]
    a1 = jnp.maximum(a1, 0.0).astype(jnp.bfloat16)
    acc3 = jnp.dot(a1, w3_ref[...], preferred_element_type=jnp.float32)
    y3_ref[...] = acc3.astype(y3_ref.dtype)
    st3_ref[...] = jnp.concatenate(
        [jnp.sum(acc3, axis=0, keepdims=True),
         jnp.sum(acc3 * acc3, axis=0, keepdims=True)], axis=0)[None]

    accs = jax.lax.dot_general(
        a2_ref[0], wsc_ref[...],
        dimension_numbers=(((0,), (0,)), ((), ())),
        preferred_element_type=jnp.float32)
    ysc_ref[...] = accs.astype(ysc_ref.dtype)
    stsc_ref[...] = jnp.concatenate(
        [jnp.sum(accs, axis=0, keepdims=True),
         jnp.sum(accs * accs, axis=0, keepdims=True)], axis=0)[None]


def _final_kernel(y3_ref, ysc_ref, pool_ref, wf1_ref, wf2_ref,
                  s3_ref, h3_ref, ssc_ref, hsc_ref, out_ref):
    g = jnp.dot(pool_ref[0], wf1_ref[...], preferred_element_type=jnp.float32)
    g = jnp.maximum(g, 0.0)
    g = jnp.dot(g, wf2_ref[...], preferred_element_type=jnp.float32)
    g = 1.0 / (1.0 + jnp.exp(-g))

    y = y3_ref[0].astype(jnp.float32) * s3_ref[...] + h3_ref[...]
    sc = ysc_ref[0].astype(jnp.float32) * ssc_ref[...] + hsc_ref[...]
    out_ref[0] = jnp.transpose(jnp.maximum(y * g + sc, 0.0))


def _bn_fold(stats, m_true, gamma, beta):
    tot = jnp.sum(stats, axis=0)
    mean = tot[0] / float(m_true)
    var = jnp.maximum(tot[1] / float(m_true) - mean * mean, 0.0)
    inv_std = jax.lax.rsqrt(var + _EPS)
    scale = gamma.astype(jnp.float32) * inv_std
    shift = beta.astype(jnp.float32) - mean * scale
    return scale, shift


def kernel(x, w1, g1, b1, w2, g2, b2, w3, g3, b3, w_sc, g_sc, b_sc,
           w_fc1, w_fc2):
    n, cin, h, w = x.shape
    c_mid = w1.shape[0]
    c_out = w3.shape[0]
    hid = w_fc1.shape[0]
    oh, ow = h // 2, w // 2
    m1 = n * h * w
    m2 = n * oh * ow
    mb = oh * ow

    hw = h * w
    w1m = jnp.transpose(w1.reshape(c_mid, cin)).astype(jnp.bfloat16)
    raw1, st1 = pl.pallas_call(
        _conv1_kernel,
        grid=(n,),
        in_specs=[
            pl.BlockSpec((1, cin, hw), lambda i: (i, 0, 0)),
            pl.BlockSpec((cin, c_mid), lambda i: (0, 0)),
        ],
        out_specs=[
            pl.BlockSpec((hw, c_mid), lambda i: (i, 0)),
            pl.BlockSpec((1, 2, c_mid), lambda i: (i, 0, 0)),
        ],
        out_shape=(
            jax.ShapeDtypeStruct((m1, c_mid), jnp.bfloat16),
            jax.ShapeDtypeStruct((n, 2, c_mid), jnp.float32),
        ),
        compiler_params=pltpu.CompilerParams(
            dimension_semantics=("parallel",)),
    )(x.reshape(n, cin, hw), w1m)
    sc1, sh1 = _bn_fold(st1, m1, g1, b1)

    hp = h // 2 + 1
    xn = raw1.reshape(n, h, w, c_mid).astype(jnp.float32) * sc1 + sh1
    xn = jnp.maximum(xn, 0.0).astype(jnp.bfloat16)
    xp = jnp.pad(xn, ((0, 0), (1, 1), (1, 1), (0, 0)))
    xps = xp.reshape(n, hp, 2, hp, 2, c_mid).transpose(0, 2, 4, 1, 3, 5)
    w2m = jnp.transpose(w2, (2, 3, 1, 0)).reshape(9, c_mid, c_mid)
    w2m = w2m.astype(jnp.bfloat16)
    raw2, st2 = pl.pallas_call(
        functools.partial(_conv2_kernel, oh=oh, mb=mb),
        grid=(n,),
        in_specs=[
            pl.BlockSpec((1, 2, 2, hp, hp, c_mid),
                         lambda i: (i, 0, 0, 0, 0, 0)),
            pl.BlockSpec((9, c_mid, c_mid), lambda i: (0, 0, 0)),
        ],
        out_specs=[
            pl.BlockSpec((mb, c_mid), lambda i: (i, 0)),
            pl.BlockSpec((1, 2, c_mid), lambda i: (i, 0, 0)),
        ],
        out_shape=(
            jax.ShapeDtypeStruct((m2, c_mid), jnp.bfloat16),
            jax.ShapeDtypeStruct((n, 2, c_mid), jnp.float32),
        ),
        compiler_params=pltpu.CompilerParams(
            dimension_semantics=("parallel",)),
    )(xps, w2m)
    sc2, sh2 = _bn_fold(st2, m2, g2, b2)

    x_sc = x[:, :, ::2, ::2].astype(jnp.bfloat16).reshape(n, cin, mb)
    w3m = jnp.transpose(w3.reshape(c_out, c_mid)).astype(jnp.bfloat16)
    wscm = jnp.transpose(w_sc.reshape(c_out, cin)).astype(jnp.bfloat16)
    raw3, rawsc, st3, stsc = pl.pallas_call(
        _conv3_sc_kernel,
        grid=(n,),
        in_specs=[
            pl.BlockSpec((mb, c_mid), lambda i: (i, 0)),
            pl.BlockSpec((1, c_mid), lambda i: (0, 0)),
            pl.BlockSpec((1, c_mid), lambda i: (0, 0)),
            pl.BlockSpec((1, cin, mb), lambda i: (i, 0, 0)),
            pl.BlockSpec((c_mid, c_out), lambda i: (0, 0)),
            pl.BlockSpec((cin, c_out), lambda i: (0, 0)),
        ],
        out_specs=[
            pl.BlockSpec((mb, c_out), lambda i: (i, 0)),
            pl.BlockSpec((mb, c_out), lambda i: (i, 0)),
            pl.BlockSpec((1, 2, c_out), lambda i: (i, 0, 0)),
            pl.BlockSpec((1, 2, c_out), lambda i: (i, 0, 0)),
        ],
        out_shape=(
            jax.ShapeDtypeStruct((m2, c_out), jnp.bfloat16),
            jax.ShapeDtypeStruct((m2, c_out), jnp.bfloat16),
            jax.ShapeDtypeStruct((n, 2, c_out), jnp.float32),
            jax.ShapeDtypeStruct((n, 2, c_out), jnp.float32),
        ),
        compiler_params=pltpu.CompilerParams(
            dimension_semantics=("parallel",)),
    )(raw2, sc2.reshape(1, c_mid), sh2.reshape(1, c_mid), x_sc, w3m, wscm)

    sc3, sh3 = _bn_fold(st3, m2, g3, b3)
    sccut, shcut = _bn_fold(stsc, m2, g_sc, b_sc)

    pooled = st3[:, 0, :] / float(mb) * sc3 + sh3
    wf1 = jnp.pad(jnp.transpose(w_fc1), ((0, 0), (0, 128 - hid)))
    wf2 = jnp.pad(jnp.transpose(w_fc2), ((0, 128 - hid), (0, 0)))

    out = pl.pallas_call(
        _final_kernel,
        grid=(n,),
        in_specs=[
            pl.BlockSpec((1, mb, c_out), lambda i: (i, 0, 0)),
            pl.BlockSpec((1, mb, c_out), lambda i: (i, 0, 0)),
            pl.BlockSpec((1, 1, c_out), lambda i: (i, 0, 0)),
            pl.BlockSpec((cin, 128), lambda i: (0, 0)),
            pl.BlockSpec((128, c_out), lambda i: (0, 0)),
            pl.BlockSpec((1, c_out), lambda i: (0, 0)),
            pl.BlockSpec((1, c_out), lambda i: (0, 0)),
            pl.BlockSpec((1, c_out), lambda i: (0, 0)),
            pl.BlockSpec((1, c_out), lambda i: (0, 0)),
        ],
        out_specs=pl.BlockSpec((1, c_out, mb), lambda i: (i, 0, 0)),
        out_shape=jax.ShapeDtypeStruct((n, c_out, mb), jnp.float32),
        compiler_params=pltpu.CompilerParams(
            dimension_semantics=("parallel",)),
    )(raw3.reshape(n, mb, c_out), rawsc.reshape(n, mb, c_out),
      pooled.reshape(n, 1, c_out), wf1, wf2,
      sc3.reshape(1, c_out), sh3.reshape(1, c_out),
      sccut.reshape(1, c_out), shcut.reshape(1, c_out))

    return out.reshape(n, c_out, oh, ow)
